# baseline (speedup 1.0000x reference)
import functools

import jax
import jax.numpy as jnp
from jax import lax
from jax.experimental import pallas as pl
from jax.experimental.pallas import tpu as pltpu
from jax.experimental.pallas import tpu_sc as plsc

EMB_DIM = 64
NUM_WORKERS = 32
NB = 16
IDX_PAD = 64
HIST_PAD = 56


def _make_gather(n_batch: int, hist: int):
    b_per_w = n_batch // NUM_WORKERS
    n_chunks = b_per_w // NB
    mesh = plsc.VectorSubcoreMesh(core_axis_name="c", subcore_axis_name="s")

    @functools.partial(
        pl.kernel,
        mesh=mesh,
        out_type=jax.ShapeDtypeStruct((n_batch, hist, EMB_DIM), jnp.float32),
        scratch_types=[
            pltpu.VMEM((NB, IDX_PAD), jnp.int32),
            pltpu.VMEM((NB, HIST_PAD, EMB_DIM), jnp.float32),
            pltpu.SemaphoreType.DMA,
            pltpu.SemaphoreType.DMA,
        ],
        compiler_params=pltpu.CompilerParams(use_tc_tiling_on_sc=False),
    )
    def gather_kernel(table_hbm, idx_hbm, out_hbm, idx_v, rows_v, sem_g, sem_w):
        wid = lax.axis_index("s") * 2 + lax.axis_index("c")
        base = wid * b_per_w

        def body(c, carry):
            off = base + c * NB
            pltpu.sync_copy(idx_hbm.at[pl.ds(off, NB)], idx_v)
            for i in range(NB):
                pltpu.async_copy(
                    table_hbm.at[idx_v.at[i, pl.ds(0, HIST_PAD)]],
                    rows_v.at[i],
                    sem_g,
                )
            for i in range(NB):
                pltpu.make_async_copy(
                    table_hbm.at[idx_v.at[i, pl.ds(0, HIST_PAD)]],
                    rows_v.at[i],
                    sem_g,
                ).wait()
            for i in range(NB):
                pltpu.async_copy(
                    rows_v.at[i, pl.ds(0, hist)], out_hbm.at[off + i], sem_w
                )
            for i in range(NB):
                pltpu.make_async_copy(
                    rows_v.at[i, pl.ds(0, hist)], out_hbm.at[off + i], sem_w
                ).wait()
            return carry

        lax.fori_loop(0, n_chunks, body, 0)

    return gather_kernel


def kernel(batch, table):
    b, h = batch.shape
    filler = (
        jnp.arange(b, dtype=jnp.int32)[:, None] * 1009
        + jnp.arange(IDX_PAD - h, dtype=jnp.int32)[None, :] * 131071
    ) % table.shape[0]
    batch_pad = jnp.concatenate([batch, filler], axis=1)
    return _make_gather(b, h)(table, batch_pad)


# speedup vs baseline: 1.0010x; 1.0010x over previous
import functools

import jax
import jax.numpy as jnp
from jax import lax
from jax.experimental import pallas as pl
from jax.experimental.pallas import tpu as pltpu
from jax.experimental.pallas import tpu_sc as plsc

EMB_DIM = 64
NUM_WORKERS = 32
NB = 16
IDX_PAD = 64
HIST_PAD = 56


def _make_gather(n_batch: int, hist: int):
    b_per_w = n_batch // NUM_WORKERS
    n_chunks = b_per_w // NB
    mesh = plsc.VectorSubcoreMesh(core_axis_name="c", subcore_axis_name="s")

    @functools.partial(
        pl.kernel,
        mesh=mesh,
        out_type=jax.ShapeDtypeStruct((n_batch, hist, EMB_DIM), jnp.float32),
        scratch_types=[
            pltpu.VMEM((NB, IDX_PAD), jnp.int32),
            pltpu.VMEM((NB, HIST_PAD, EMB_DIM), jnp.float32),
            pltpu.SemaphoreType.DMA,
            pltpu.SemaphoreType.DMA,
        ],
        compiler_params=pltpu.CompilerParams(use_tc_tiling_on_sc=False),
    )
    def gather_kernel(idx_hbm, table_hbm, out_hbm, idx_v, rows_v, sem_g, sem_w):
        wid = lax.axis_index("s") * 2 + lax.axis_index("c")
        base = wid * b_per_w

        def body(c, carry):
            off = base + c * NB
            pltpu.sync_copy(idx_hbm.at[pl.ds(off, NB)], idx_v)
            for i in range(NB):
                pltpu.async_copy(
                    table_hbm.at[idx_v.at[i, pl.ds(0, HIST_PAD)]],
                    rows_v.at[i],
                    sem_g,
                )
            for i in range(NB):
                pltpu.make_async_copy(
                    table_hbm.at[idx_v.at[i, pl.ds(0, HIST_PAD)]],
                    rows_v.at[i],
                    sem_g,
                ).wait()
            for i in range(NB):
                pltpu.async_copy(
                    rows_v.at[i, pl.ds(0, hist)], out_hbm.at[off + i], sem_w
                )
            for i in range(NB):
                pltpu.make_async_copy(
                    rows_v.at[i, pl.ds(0, hist)], out_hbm.at[off + i], sem_w
                ).wait()
            return carry

        lax.fori_loop(0, n_chunks, body, 0)

    return gather_kernel


def kernel(batch, table):
    b, h = batch.shape
    filler = (
        jnp.arange(b, dtype=jnp.int32)[:, None] * 1009
        + jnp.arange(IDX_PAD - h, dtype=jnp.int32)[None, :] * 131071
    ) % table.shape[0]
    batch_pad = jnp.concatenate([batch, filler], axis=1)
    return _make_gather(b, h)(batch_pad, table)


# speedup vs baseline: 1.0342x; 1.0332x over previous
import functools

import jax
import jax.numpy as jnp
from jax import lax
from jax.experimental import pallas as pl
from jax.experimental.pallas import tpu as pltpu
from jax.experimental.pallas import tpu_sc as plsc

EMB_DIM = 64
NUM_WORKERS = 32
CHUNK = 800
NBUF = 2


def _make_gather(batch_flat_len: int):
    b_per_w = batch_flat_len // NUM_WORKERS
    n_chunks = b_per_w // CHUNK
    assert n_chunks % NBUF == 0
    mesh = plsc.VectorSubcoreMesh(core_axis_name="c", subcore_axis_name="s")

    @functools.partial(
        pl.kernel,
        mesh=mesh,
        out_type=jax.ShapeDtypeStruct((batch_flat_len, EMB_DIM), jnp.float32),
        scratch_types=[
            pltpu.VMEM((b_per_w,), jnp.int32),
            pltpu.VMEM((NBUF, CHUNK, EMB_DIM), jnp.float32),
            [pltpu.SemaphoreType.DMA] * NBUF,
            [pltpu.SemaphoreType.DMA] * NBUF,
        ],
        compiler_params=pltpu.CompilerParams(use_tc_tiling_on_sc=False),
    )
    def gather_kernel(table_hbm, idx_hbm, out_hbm, idx_v, rows_v, sems_g, sems_o):
        wid = lax.axis_index("s") * 2 + lax.axis_index("c")
        base = wid * b_per_w
        pltpu.sync_copy(idx_hbm.at[pl.ds(base, b_per_w)], idx_v)

        def g_desc(c, b):
            off = pl.multiple_of(c * CHUNK, 8)
            return pltpu.make_async_copy(
                table_hbm.at[idx_v.at[pl.ds(off, CHUNK)]], rows_v.at[b], sems_g[b]
            )

        def o_desc(c, b):
            off = pl.multiple_of(c * CHUNK, 8)
            return pltpu.make_async_copy(
                rows_v.at[b], out_hbm.at[pl.ds(base + off, CHUNK)], sems_o[b]
            )

        g_desc(0, 0).start()

        def pair_body(p, carry):
            for b in range(NBUF):
                c = NBUF * p + b
                g_desc(c, b).wait()
                @pl.when(c >= 1)
                def _():
                    o_desc(c - 1, 1 - b).wait()

                @pl.when(c + 1 < n_chunks)
                def _():
                    g_desc(c + 1, 1 - b).start()

                o_desc(c, b).start()
            return carry

        lax.fori_loop(0, n_chunks // NBUF, pair_body, 0)
        o_desc(n_chunks - 1, (n_chunks - 1) % NBUF).wait()

    return gather_kernel


def kernel(batch, table):
    b, h = batch.shape
    flat_idx = batch.reshape(b * h)
    out = _make_gather(b * h)(table, flat_idx)
    return out.reshape(b, h, EMB_DIM)
